# baseline (speedup 1.0000x reference)
import functools

import jax
import jax.numpy as jnp
from jax.experimental import pallas as pl
from jax.experimental.pallas import tpu as pltpu


def _fused_kernel(x_ref, w_ref, g_ref, b_ref, o_ref, y_scr, st_scr,
                  *, K, L, R0, R1, nG0, count, eps):
    step = pl.program_id(0)
    pad = K // 2
    cin = x_ref.shape[1]

    @pl.when(step < nG0)
    def _phase0():
        z = jnp.zeros((cin, pad), jnp.bfloat16)
        H = 4 if R0 % 4 == 0 else (2 if R0 % 2 == 0 else 1)
        RH = R0 // H
        s_parts, s2_parts = [], []
        for h in range(H):
            cols = []
            for r in range(h * RH, (h + 1) * RH):
                xp = jnp.concatenate(
                    [z, x_ref[r].astype(jnp.bfloat16), z], axis=1)
                cols.append(jnp.concatenate(
                    [xp[:, k:k + L] for k in range(K)], axis=0))
            im2col = jnp.concatenate(cols, axis=1)

            acc = jax.lax.dot_general(
                w_ref[...], im2col,
                dimension_numbers=(((1,), (0,)), ((), ())),
                preferred_element_type=jnp.float32)
            acc = jnp.maximum(acc, 0.0)

            for i in range(RH):
                y_scr[step * R0 + h * RH + i] = (
                    acc[:, i * L:(i + 1) * L].astype(y_scr.dtype))
            s_parts.append(jnp.sum(acc, axis=1))
            s2_parts.append(jnp.sum(acc * acc, axis=1))

        part = jnp.stack([sum(s_parts), sum(s2_parts)], axis=0)

        @pl.when(step == 0)
        def _init():
            st_scr[...] = part

        @pl.when(step > 0)
        def _accum():
            st_scr[...] = st_scr[...] + part

    @pl.when(step >= nG0)
    def _phase1():
        gp = step - nG0

        @pl.when(gp == 0)
        def _fold():
            mean = st_scr[0] / count
            var = st_scr[1] / count - mean * mean
            inv = jax.lax.rsqrt(var + eps)
            scale = g_ref[0] * inv
            st_scr[...] = jnp.stack([scale, b_ref[0] - mean * scale], axis=0)

        scale = st_scr[0][None, :, None]
        shift = st_scr[1][None, :, None]
        y = y_scr[pl.ds(gp * R1, R1)].astype(jnp.float32)
        o_ref[...] = (y * scale + shift).astype(o_ref.dtype)


def _pick_rows(b, cands):
    for r in cands:
        if b % r == 0:
            return r
    return 1


def kernel(x, weight, gamma, beta, *, eps=1e-5):
    B, Cin, L = x.shape
    Cout, _, K = weight.shape
    R0 = _pick_rows(B, (8, 4, 2))
    R1 = _pick_rows(B, (4, 2))
    nG0 = B // R0
    nG1 = B // R1

    w = jnp.transpose(weight, (0, 2, 1)).reshape(Cout, K * Cin).astype(jnp.bfloat16)

    fused = functools.partial(_fused_kernel, K=K, L=L, R0=R0, R1=R1, nG0=nG0,
                              count=float(B * L), eps=eps)
    out = pl.pallas_call(
        fused,
        out_shape=jax.ShapeDtypeStruct((B, Cout, L), x.dtype),
        grid=(nG0 + nG1,),
        in_specs=[
            pl.BlockSpec((R0, Cin, L), lambda s: (jnp.minimum(s, nG0 - 1), 0, 0)),
            pl.BlockSpec((Cout, K * Cin), lambda s: (0, 0)),
            pl.BlockSpec((1, Cout), lambda s: (0, 0)),
            pl.BlockSpec((1, Cout), lambda s: (0, 0)),
        ],
        out_specs=pl.BlockSpec(
            (R1, Cout, L), lambda s: (jnp.maximum(s - nG0, 0), 0, 0)),
        scratch_shapes=[
            pltpu.VMEM((B, Cout, L), jnp.bfloat16),
            pltpu.VMEM((2, Cout), jnp.float32),
        ],
        compiler_params=pltpu.CompilerParams(
            dimension_semantics=("arbitrary",),
            vmem_limit_bytes=64 * 1024 * 1024),
    )(x, w, gamma.reshape(1, Cout), beta.reshape(1, Cout))
    return out


# speedup vs baseline: 1.0152x; 1.0152x over previous
import functools

import jax
import jax.numpy as jnp
from jax.experimental import pallas as pl
from jax.experimental.pallas import tpu as pltpu


def _fused_kernel(x_ref, w_ref, g_ref, b_ref, o_ref, y_scr, st_scr,
                  *, K, L, R0, R1, nG0, count, eps):
    step = pl.program_id(0)
    pad = K // 2
    cin = x_ref.shape[1]

    @pl.when(step < nG0)
    def _phase0():
        z = jnp.zeros((cin, pad), jnp.bfloat16)
        H = 2 if R0 % 2 == 0 else 1
        RH = R0 // H
        s_parts, s2_parts = [], []
        for h in range(H):
            cols = []
            for r in range(h * RH, (h + 1) * RH):
                xp = jnp.concatenate(
                    [z, x_ref[r].astype(jnp.bfloat16), z], axis=1)
                cols.append(jnp.concatenate(
                    [xp[:, k:k + L] for k in range(K)], axis=0))
            im2col = jnp.concatenate(cols, axis=1)

            acc = jax.lax.dot_general(
                w_ref[...], im2col,
                dimension_numbers=(((1,), (0,)), ((), ())),
                preferred_element_type=jnp.float32)
            acc = jnp.maximum(acc, 0.0)

            for i in range(RH):
                y_scr[step * R0 + h * RH + i] = (
                    acc[:, i * L:(i + 1) * L].astype(y_scr.dtype))
            s_parts.append(jnp.sum(acc, axis=1))
            s2_parts.append(jnp.sum(acc * acc, axis=1))

        part = jnp.stack([sum(s_parts), sum(s2_parts)], axis=0)

        @pl.when(step == 0)
        def _init():
            st_scr[...] = part

        @pl.when(step > 0)
        def _accum():
            st_scr[...] = st_scr[...] + part

    @pl.when(step >= nG0)
    def _phase1():
        gp = step - nG0

        @pl.when(gp == 0)
        def _fold():
            mean = st_scr[0] / count
            var = st_scr[1] / count - mean * mean
            inv = jax.lax.rsqrt(var + eps)
            scale = g_ref[0] * inv
            st_scr[...] = jnp.stack([scale, b_ref[0] - mean * scale], axis=0)

        scale = st_scr[0][None, :, None]
        shift = st_scr[1][None, :, None]
        y = y_scr[pl.ds(gp * R1, R1)].astype(jnp.float32)
        o_ref[...] = (y * scale + shift).astype(o_ref.dtype)


def _pick_rows(b, cands):
    for r in cands:
        if b % r == 0:
            return r
    return 1


def kernel(x, weight, gamma, beta, *, eps=1e-5):
    B, Cin, L = x.shape
    Cout, _, K = weight.shape
    R0 = _pick_rows(B, (8, 4, 2))
    R1 = _pick_rows(B, (4, 2))
    nG0 = B // R0
    nG1 = B // R1

    w = jnp.transpose(weight, (0, 2, 1)).reshape(Cout, K * Cin).astype(jnp.bfloat16)

    fused = functools.partial(_fused_kernel, K=K, L=L, R0=R0, R1=R1, nG0=nG0,
                              count=float(B * L), eps=eps)
    out = pl.pallas_call(
        fused,
        out_shape=jax.ShapeDtypeStruct((B, Cout, L), x.dtype),
        grid=(nG0 + nG1,),
        in_specs=[
            pl.BlockSpec((R0, Cin, L), lambda s: (jnp.minimum(s, nG0 - 1), 0, 0)),
            pl.BlockSpec((Cout, K * Cin), lambda s: (0, 0)),
            pl.BlockSpec((1, Cout), lambda s: (0, 0)),
            pl.BlockSpec((1, Cout), lambda s: (0, 0)),
        ],
        out_specs=pl.BlockSpec(
            (R1, Cout, L), lambda s: (jnp.maximum(s - nG0, 0), 0, 0)),
        scratch_shapes=[
            pltpu.VMEM((B, Cout, L), jnp.bfloat16),
            pltpu.VMEM((2, Cout), jnp.float32),
        ],
        compiler_params=pltpu.CompilerParams(
            dimension_semantics=("arbitrary",),
            vmem_limit_bytes=64 * 1024 * 1024),
    )(x, w, gamma.reshape(1, Cout), beta.reshape(1, Cout))
    return out
